# baseline (speedup 1.0000x reference)
import functools

import jax
import jax.numpy as jnp
from jax import lax
from jax.experimental import pallas as pl
from jax.experimental.pallas import tpu as pltpu
from jax.experimental.pallas import tpu_sc as plsc

_F32 = jnp.float32
_DEF = jax.lax.Precision.DEFAULT
_BIG = 1e30


def _erf(x):
    a1, a2, a3, a4, a5 = 0.254829592, -0.284496736, 1.421413741, -1.453152027, 1.061405429
    p = 0.3275911
    s = jnp.sign(x)
    ax = jnp.abs(x)
    t = 1.0 / (1.0 + p * ax)
    poly = ((((a5 * t + a4) * t + a3) * t + a2) * t + a1) * t
    return s * (1.0 - poly * jnp.exp(-ax * ax))


def _top4_gather(c_tile, cT, emb, n_cand, k):
    rowsq = jnp.sum(c_tile * c_tile, axis=1, keepdims=True)
    colsq = jnp.sum(cT * cT, axis=0, keepdims=True)
    rc = jax.lax.dot_general(c_tile, cT, (((1,), (0,)), ((), ())), precision=_DEF)
    d2 = jnp.maximum(rowsq + colsq - 2.0 * rc, 0.0)
    T = c_tile.shape[0]
    iota = jax.lax.broadcasted_iota(jnp.int32, (T, n_cand), 1).astype(_F32)
    feats = []
    for _ in range(k):
        m = jnp.min(d2, axis=1, keepdims=True)
        cand = jnp.where(d2 == m, iota, float(n_cand))
        idxk = jnp.min(cand, axis=1, keepdims=True)
        hit = iota == idxk
        oh = hit.astype(_F32)
        if emb.shape[1] == n_cand:
            nattr = jax.lax.dot_general(oh, emb, (((1,), (1,)), ((), ())), precision=_DEF)
        else:
            nattr = jax.lax.dot_general(oh, emb, (((1,), (0,)), ((), ())), precision=_DEF)
        d2 = jnp.where(hit, _BIG, d2)
        feats.append(jnp.sqrt(m + 1e-10))
        feats.append(nattr)
    return feats


def _atom_kernel(c_tile_ref, cT_ref, attr_ref, idxres_ref, tblT_ref,
                 feat_ref, idxg_ref, *, TA, La, L, K):
    b = pl.program_id(0)
    c_tile = c_tile_ref[0]
    cT = cT_ref[0]
    attr = attr_ref[0]
    idxres = idxres_ref[0]
    tblT = tblT_ref[...]

    embT = jnp.zeros((tblT.shape[0], La), _F32)
    for v in range(tblT.shape[1]):
        embT = embT + tblT[:, v:v + 1] * (attr == float(v)).astype(_F32)

    feats = _top4_gather(c_tile, cT, embT, La, K)
    pad = feat_ref.shape[2] - 52
    feats.append(jnp.zeros((TA, pad), _F32))
    feat_ref[...] = jnp.concatenate(feats, axis=1)[None]
    idxg_ref[...] = (idxres + jnp.float32(L) * b.astype(_F32))[None].astype(jnp.int32)


def _make_sc_scatter(n_rows, n_atoms, width, chunk):
    NC, NS = 2, 16
    n_j = chunk // 128
    rows_per_sub = n_rows // NS
    wb_per_sub = (n_rows // NC) // NS
    mesh = plsc.VectorSubcoreMesh(core_axis_name="c", subcore_axis_name="s")

    @functools.partial(
        pl.kernel, mesh=mesh,
        out_type=jax.ShapeDtypeStruct((n_rows, width), _F32),
        scratch_types=[
            [pltpu.VMEM((128,), jnp.int32) for _ in range(n_j)],
            pltpu.VMEM((chunk, width), _F32),
            pltpu.VMEM((rows_per_sub, width), _F32),
            pltpu.VMEM_SHARED((n_rows, width), _F32),
        ],
    )
    def sc_scatter(feat_hbm, idx_hbm, out_hbm, idx_refs, feat_v, zbuf, acc):
        c = lax.axis_index("c")
        s = lax.axis_index("s")
        wid = c * NS + s
        base = wid * chunk

        def zbody(i, carry):
            for j in range(width // 16):
                zbuf[i, pl.ds(j * 16, 16)] = jnp.zeros((16,), _F32)
            return carry

        lax.fori_loop(0, rows_per_sub, zbody, 0)
        pltpu.sync_copy(zbuf, acc.at[pl.ds(s * rows_per_sub, rows_per_sub)])
        for j in range(n_j):
            pltpu.sync_copy(idx_hbm.at[pl.ds(base + j * 128, 128)], idx_refs[j])
        pltpu.sync_copy(feat_hbm.at[pl.ds(base, chunk)], feat_v)
        plsc.subcore_barrier()
        for j in range(n_j):
            pltpu.sync_copy(feat_v.at[pl.ds(j * 128, 128)],
                            acc.at[idx_refs[j]], add=True)
        plsc.subcore_barrier()
        wb = c * (n_rows // NC) + s * wb_per_sub
        pltpu.sync_copy(acc.at[pl.ds(wb, wb_per_sub)],
                        out_hbm.at[pl.ds(wb, wb_per_sub)])

    return sc_scatter


def _res_kernel(c_tile_ref, cT_ref, attr_ref, gath_ref, Waa_ref, baa_ref,
                W1_ref, b1_ref, g_ref, be_ref, W2_ref, b2_ref, out_ref,
                *, TR, L, K, DPAD):
    c_tile = c_tile_ref[0]
    cT = cT_ref[0]
    attr = attr_ref[0]
    gath = gath_ref[0][:, :52]

    emb_aa = jax.lax.dot_general(attr, Waa_ref[...], (((1,), (0,)), ((), ())),
                                 precision=_DEF) + baa_ref[...]
    emb = jnp.concatenate([emb_aa, gath], axis=1)

    feats = _top4_gather(c_tile, cT, emb, L, K)
    feats.append(jnp.zeros((c_tile.shape[0], DPAD - 276), _F32))
    feat = jnp.concatenate(feats, axis=1)

    h = jax.lax.dot_general(feat, W1_ref[...], (((1,), (0,)), ((), ())),
                            precision=_DEF) + b1_ref[...]
    mu = jnp.mean(h, axis=1, keepdims=True)
    var = jnp.mean((h - mu) ** 2, axis=1, keepdims=True)
    hn = (h - mu) / jnp.sqrt(var + 1e-5) * g_ref[...] + be_ref[...]
    ge = 0.5 * hn * (1.0 + _erf(hn * 0.7071067811865476))
    logits = jax.lax.dot_general(ge, W2_ref[...], (((1,), (0,)), ((), ())),
                                 precision=_DEF) + b2_ref[...]
    out_ref[...] = logits[None]


def kernel(coord_aa, attr_aa, triplets_aa, indices_aa, coord_atom, attr_atom,
           triplets_atom, indices_atom, W_aa, b_aa, atom_table, W1, b1, gamma,
           beta, W2, b2):
    B, L, _ = coord_aa.shape
    La = coord_atom.shape[1]
    K = 4
    TA = 256
    TR = 256
    DPAD = 384
    WID = 128

    c_atom = coord_atom.astype(_F32)
    c_atomT = jnp.transpose(c_atom, (0, 2, 1))
    attr_f = attr_atom.astype(_F32)[:, None, :]
    idxres_f = indices_atom[..., 0].astype(_F32)[:, None, :]
    tblT = atom_table.T.astype(_F32)

    atom_feat, idx_g = pl.pallas_call(
        functools.partial(_atom_kernel, TA=TA, La=La, L=L, K=K),
        grid=(B, La // TA),
        in_specs=[
            pl.BlockSpec((1, TA, 3), lambda b, t: (b, t, 0)),
            pl.BlockSpec((1, 3, La), lambda b, t: (b, 0, 0)),
            pl.BlockSpec((1, 1, La), lambda b, t: (b, 0, 0)),
            pl.BlockSpec((1, 1, TA), lambda b, t: (b, 0, t)),
            pl.BlockSpec((12, 13), lambda b, t: (0, 0)),
        ],
        out_specs=[
            pl.BlockSpec((1, TA, WID), lambda b, t: (b, t, 0)),
            pl.BlockSpec((1, 1, TA), lambda b, t: (b, 0, t)),
        ],
        out_shape=[
            jax.ShapeDtypeStruct((B, La, WID), _F32),
            jax.ShapeDtypeStruct((B, 1, La), jnp.int32),
        ],
    )(c_atom, c_atomT, attr_f, idxres_f, tblT)

    n_tot = B * La
    sc_scatter = _make_sc_scatter(B * L, n_tot, WID, n_tot // 32)
    gathered = sc_scatter(atom_feat.reshape(n_tot, WID),
                          idx_g.reshape(n_tot))
    gathered = gathered.reshape(B, L, WID)

    c_aa = coord_aa.astype(_F32)
    c_aaT = jnp.transpose(c_aa, (0, 2, 1))
    W1p = W1[:DPAD].astype(_F32)

    out3 = pl.pallas_call(
        functools.partial(_res_kernel, TR=TR, L=L, K=K, DPAD=DPAD),
        grid=(B, L // TR),
        in_specs=[
            pl.BlockSpec((1, TR, 3), lambda b, t: (b, t, 0)),
            pl.BlockSpec((1, 3, L), lambda b, t: (b, 0, 0)),
            pl.BlockSpec((1, L, 20), lambda b, t: (b, 0, 0)),
            pl.BlockSpec((1, L, WID), lambda b, t: (b, 0, 0)),
            pl.BlockSpec((20, 16), lambda b, t: (0, 0)),
            pl.BlockSpec((1, 16), lambda b, t: (0, 0)),
            pl.BlockSpec((DPAD, 256), lambda b, t: (0, 0)),
            pl.BlockSpec((1, 256), lambda b, t: (0, 0)),
            pl.BlockSpec((1, 256), lambda b, t: (0, 0)),
            pl.BlockSpec((1, 256), lambda b, t: (0, 0)),
            pl.BlockSpec((256, 1), lambda b, t: (0, 0)),
            pl.BlockSpec((1, 1), lambda b, t: (0, 0)),
        ],
        out_specs=pl.BlockSpec((1, TR, 1), lambda b, t: (b, t, 0)),
        out_shape=jax.ShapeDtypeStruct((B, L, 1), _F32),
    )(c_aa, c_aaT, attr_aa.astype(_F32), gathered, W_aa.astype(_F32),
      b_aa.astype(_F32)[None], W1p, b1.astype(_F32)[None],
      gamma.astype(_F32)[None], beta.astype(_F32)[None], W2.astype(_F32),
      b2.astype(_F32)[None])

    return out3[..., 0]


# speedup vs baseline: 1.1002x; 1.1002x over previous
import functools

import jax
import jax.numpy as jnp
from jax import lax
from jax.experimental import pallas as pl
from jax.experimental.pallas import tpu as pltpu
from jax.experimental.pallas import tpu_sc as plsc

_F32 = jnp.float32
_DEF = jax.lax.Precision.DEFAULT
_BIG = 1e30


def _erf(x):
    a1, a2, a3, a4, a5 = 0.254829592, -0.284496736, 1.421413741, -1.453152027, 1.061405429
    p = 0.3275911
    s = jnp.sign(x)
    ax = jnp.abs(x)
    t = 1.0 / (1.0 + p * ax)
    poly = ((((a5 * t + a4) * t + a3) * t + a2) * t + a1) * t
    return s * (1.0 - poly * jnp.exp(-ax * ax))


def _top4_gather(c_tile, cT, emb, n_cand, k):
    rowsq = jnp.sum(c_tile * c_tile, axis=1, keepdims=True)
    colsq = jnp.sum(cT * cT, axis=0, keepdims=True)
    rc = jax.lax.dot_general(c_tile, cT, (((1,), (0,)), ((), ())), precision=_DEF)
    d2 = jnp.maximum(rowsq + colsq - 2.0 * rc, 0.0)
    T = c_tile.shape[0]
    iota = jax.lax.broadcasted_iota(jnp.int32, (T, n_cand), 1).astype(_F32)
    feats = []
    for _ in range(k):
        m = jnp.min(d2, axis=1, keepdims=True)
        cand = jnp.where(d2 == m, iota, float(n_cand))
        idxk = jnp.min(cand, axis=1, keepdims=True)
        hit = iota == idxk
        oh = hit.astype(_F32)
        if emb.shape[1] == n_cand:
            nattr = jax.lax.dot_general(oh, emb, (((1,), (1,)), ((), ())), precision=_DEF)
        else:
            nattr = jax.lax.dot_general(oh, emb, (((1,), (0,)), ((), ())), precision=_DEF)
        d2 = jnp.where(hit, _BIG, d2)
        feats.append(jnp.sqrt(m + 1e-10))
        feats.append(nattr)
    return feats


def _atom_kernel(c_tile_ref, cT_ref, attr_ref, idxres_ref, tblT_ref,
                 feat_ref, idxg_ref, *, TA, La, L, K):
    b = pl.program_id(0)
    c_tile = c_tile_ref[0]
    cT = cT_ref[0]
    attr = attr_ref[0]
    idxres = idxres_ref[0]
    tblT = tblT_ref[...]

    embT = jnp.zeros((tblT.shape[0], La), _F32)
    for v in range(tblT.shape[1]):
        embT = embT + tblT[:, v:v + 1] * (attr == float(v)).astype(_F32)

    feats = _top4_gather(c_tile, cT, embT, La, K)
    pad = feat_ref.shape[2] - 52
    feats.append(jnp.zeros((TA, pad), _F32))
    feat_ref[...] = jnp.concatenate(feats, axis=1)[None]
    idxg_ref[...] = (idxres + jnp.float32(L) * b.astype(_F32))[None].astype(jnp.int32)


def _make_sc_scatter(n_rows, n_atoms, width, chunk):
    NC, NS = 2, 16
    n_j = chunk // 128
    rows_per_sub = n_rows // NS
    wb_per_sub = (n_rows // NC) // NS
    mesh = plsc.VectorSubcoreMesh(core_axis_name="c", subcore_axis_name="s")

    @functools.partial(
        pl.kernel, mesh=mesh,
        out_type=jax.ShapeDtypeStruct((n_rows, width), _F32),
        scratch_types=[
            [pltpu.VMEM((128,), jnp.int32) for _ in range(n_j)],
            pltpu.VMEM((chunk, width), _F32),
            pltpu.VMEM((rows_per_sub, width), _F32),
            pltpu.VMEM_SHARED((n_rows, width), _F32),
        ],
    )
    def sc_scatter(feat_hbm, idx_hbm, out_hbm, idx_refs, feat_v, zbuf, acc):
        c = lax.axis_index("c")
        s = lax.axis_index("s")
        wid = c * NS + s
        base = wid * chunk

        def zbody(i, carry):
            for j in range(width // 16):
                zbuf[i, pl.ds(j * 16, 16)] = jnp.zeros((16,), _F32)
            return carry

        lax.fori_loop(0, rows_per_sub, zbody, 0)
        pltpu.sync_copy(zbuf, acc.at[pl.ds(s * rows_per_sub, rows_per_sub)])
        for j in range(n_j):
            pltpu.sync_copy(idx_hbm.at[pl.ds(base + j * 128, 128)], idx_refs[j])
        pltpu.sync_copy(feat_hbm.at[pl.ds(base, chunk)], feat_v)
        plsc.subcore_barrier()
        for j in range(n_j):
            pltpu.sync_copy(feat_v.at[pl.ds(j * 128, 128)],
                            acc.at[idx_refs[j]], add=True)
        plsc.subcore_barrier()
        wb = c * (n_rows // NC) + s * wb_per_sub
        pltpu.sync_copy(acc.at[pl.ds(wb, wb_per_sub)],
                        out_hbm.at[pl.ds(wb, wb_per_sub)])

    return sc_scatter


def _res_kernel(c_tile_ref, cT_ref, attr_ref, gath_ref, Waa_ref, baa_ref,
                W1_ref, b1_ref, g_ref, be_ref, W2_ref, b2_ref, out_ref,
                *, TR, L, K, DPAD):
    c_tile = c_tile_ref[0]
    cT = cT_ref[0]
    attr = attr_ref[0]
    gath = gath_ref[0][:, :52]

    emb_aa = jax.lax.dot_general(attr, Waa_ref[...], (((1,), (0,)), ((), ())),
                                 precision=_DEF) + baa_ref[...]
    emb = jnp.concatenate([emb_aa, gath], axis=1)

    feats = _top4_gather(c_tile, cT, emb, L, K)
    feats.append(jnp.zeros((c_tile.shape[0], DPAD - 276), _F32))
    feat = jnp.concatenate(feats, axis=1)

    h = jax.lax.dot_general(feat, W1_ref[...], (((1,), (0,)), ((), ())),
                            precision=_DEF) + b1_ref[...]
    mu = jnp.mean(h, axis=1, keepdims=True)
    var = jnp.mean((h - mu) ** 2, axis=1, keepdims=True)
    hn = (h - mu) / jnp.sqrt(var + 1e-5) * g_ref[...] + be_ref[...]
    ge = 0.5 * hn * (1.0 + _erf(hn * 0.7071067811865476))
    logits = jax.lax.dot_general(ge, W2_ref[...], (((1,), (0,)), ((), ())),
                                 precision=_DEF) + b2_ref[...]
    out_ref[...] = logits[None]


def kernel(coord_aa, attr_aa, triplets_aa, indices_aa, coord_atom, attr_atom,
           triplets_atom, indices_atom, W_aa, b_aa, atom_table, W1, b1, gamma,
           beta, W2, b2):
    B, L, _ = coord_aa.shape
    La = coord_atom.shape[1]
    K = 4
    TA = 512
    TR = 512
    DPAD = 384
    WID = 128

    c_atom = coord_atom.astype(_F32)
    c_atomT = jnp.transpose(c_atom, (0, 2, 1))
    attr_f = attr_atom.astype(_F32)[:, None, :]
    idxres_f = indices_atom[..., 0].astype(_F32)[:, None, :]
    tblT = atom_table.T.astype(_F32)

    atom_feat, idx_g = pl.pallas_call(
        functools.partial(_atom_kernel, TA=TA, La=La, L=L, K=K),
        grid=(B, La // TA),
        in_specs=[
            pl.BlockSpec((1, TA, 3), lambda b, t: (b, t, 0)),
            pl.BlockSpec((1, 3, La), lambda b, t: (b, 0, 0)),
            pl.BlockSpec((1, 1, La), lambda b, t: (b, 0, 0)),
            pl.BlockSpec((1, 1, TA), lambda b, t: (b, 0, t)),
            pl.BlockSpec((12, 13), lambda b, t: (0, 0)),
        ],
        out_specs=[
            pl.BlockSpec((1, TA, WID), lambda b, t: (b, t, 0)),
            pl.BlockSpec((1, 1, TA), lambda b, t: (b, 0, t)),
        ],
        out_shape=[
            jax.ShapeDtypeStruct((B, La, WID), _F32),
            jax.ShapeDtypeStruct((B, 1, La), jnp.int32),
        ],
    )(c_atom, c_atomT, attr_f, idxres_f, tblT)

    n_tot = B * La
    sc_scatter = _make_sc_scatter(B * L, n_tot, WID, n_tot // 32)
    gathered = sc_scatter(atom_feat.reshape(n_tot, WID),
                          idx_g.reshape(n_tot))
    gathered = gathered.reshape(B, L, WID)

    c_aa = coord_aa.astype(_F32)
    c_aaT = jnp.transpose(c_aa, (0, 2, 1))
    W1p = W1[:DPAD].astype(_F32)

    out3 = pl.pallas_call(
        functools.partial(_res_kernel, TR=TR, L=L, K=K, DPAD=DPAD),
        grid=(B, L // TR),
        in_specs=[
            pl.BlockSpec((1, TR, 3), lambda b, t: (b, t, 0)),
            pl.BlockSpec((1, 3, L), lambda b, t: (b, 0, 0)),
            pl.BlockSpec((1, L, 20), lambda b, t: (b, 0, 0)),
            pl.BlockSpec((1, L, WID), lambda b, t: (b, 0, 0)),
            pl.BlockSpec((20, 16), lambda b, t: (0, 0)),
            pl.BlockSpec((1, 16), lambda b, t: (0, 0)),
            pl.BlockSpec((DPAD, 256), lambda b, t: (0, 0)),
            pl.BlockSpec((1, 256), lambda b, t: (0, 0)),
            pl.BlockSpec((1, 256), lambda b, t: (0, 0)),
            pl.BlockSpec((1, 256), lambda b, t: (0, 0)),
            pl.BlockSpec((256, 1), lambda b, t: (0, 0)),
            pl.BlockSpec((1, 1), lambda b, t: (0, 0)),
        ],
        out_specs=pl.BlockSpec((1, TR, 1), lambda b, t: (b, t, 0)),
        out_shape=jax.ShapeDtypeStruct((B, L, 1), _F32),
    )(c_aa, c_aaT, attr_aa.astype(_F32), gathered, W_aa.astype(_F32),
      b_aa.astype(_F32)[None], W1p, b1.astype(_F32)[None],
      gamma.astype(_F32)[None], beta.astype(_F32)[None], W2.astype(_F32),
      b2.astype(_F32)[None])

    return out3[..., 0]


# speedup vs baseline: 1.1401x; 1.0363x over previous
import functools

import jax
import jax.numpy as jnp
from jax import lax
from jax.experimental import pallas as pl
from jax.experimental.pallas import tpu as pltpu
from jax.experimental.pallas import tpu_sc as plsc

_F32 = jnp.float32
_DEF = jax.lax.Precision.DEFAULT
_BIG = 1e30


def _erf(x):
    a1, a2, a3, a4, a5 = 0.254829592, -0.284496736, 1.421413741, -1.453152027, 1.061405429
    p = 0.3275911
    s = jnp.sign(x)
    ax = jnp.abs(x)
    t = 1.0 / (1.0 + p * ax)
    poly = ((((a5 * t + a4) * t + a3) * t + a2) * t + a1) * t
    return s * (1.0 - poly * jnp.exp(-ax * ax))


def _top4_gather(c_tile, cT, emb, n_cand, k):
    rowsq = jnp.sum(c_tile * c_tile, axis=1, keepdims=True)
    colsq = jnp.sum(cT * cT, axis=0, keepdims=True)
    rc = jax.lax.dot_general(c_tile, cT, (((1,), (0,)), ((), ())), precision=_DEF)
    d2 = jnp.maximum(rowsq + colsq - 2.0 * rc, 0.0)
    T = c_tile.shape[0]
    iota = jax.lax.broadcasted_iota(jnp.int32, (T, n_cand), 1).astype(_F32)
    feats = []
    for _ in range(k):
        m = jnp.min(d2, axis=1, keepdims=True)
        cand = jnp.where(d2 == m, iota, float(n_cand))
        idxk = jnp.min(cand, axis=1, keepdims=True)
        hit = iota == idxk
        oh = hit.astype(_F32)
        if emb.shape[1] == n_cand:
            nattr = jax.lax.dot_general(oh, emb, (((1,), (1,)), ((), ())), precision=_DEF)
        else:
            nattr = jax.lax.dot_general(oh, emb, (((1,), (0,)), ((), ())), precision=_DEF)
        d2 = jnp.where(hit, _BIG, d2)
        feats.append(jnp.sqrt(m + 1e-10))
        feats.append(nattr)
    return feats


def _atom_kernel(c_tile_ref, cT_ref, attr_ref, idxres_ref, tblT_ref,
                 feat_ref, idxg_ref, *, TA, La, L, K):
    b = pl.program_id(0)
    c_tile = c_tile_ref[0]
    cT = cT_ref[0]
    attr = attr_ref[0]
    idxres = idxres_ref[0]
    tblT = tblT_ref[...]

    embT = jnp.zeros((tblT.shape[0], La), _F32)
    for v in range(tblT.shape[1]):
        embT = embT + tblT[:, v:v + 1] * (attr == float(v)).astype(_F32)

    feats = _top4_gather(c_tile, cT, embT, La, K)
    pad = feat_ref.shape[2] - 52
    feats.append(jnp.zeros((TA, pad), _F32))
    feat_ref[...] = jnp.concatenate(feats, axis=1)[None]
    idxg_ref[...] = (idxres + jnp.float32(L) * b.astype(_F32))[None].astype(jnp.int32)


def _make_sc_scatter(n_rows, n_atoms, width, chunk):
    NC, NS = 2, 16
    n_j = chunk // 128
    rows_per_sub = n_rows // NS
    wb_per_sub = (n_rows // NC) // NS
    mesh = plsc.VectorSubcoreMesh(core_axis_name="c", subcore_axis_name="s")

    @functools.partial(
        pl.kernel, mesh=mesh,
        out_type=jax.ShapeDtypeStruct((n_rows, width), _F32),
        scratch_types=[
            [pltpu.VMEM((128,), jnp.int32) for _ in range(n_j)],
            pltpu.VMEM((chunk, width), _F32),
            pltpu.VMEM((rows_per_sub, width), _F32),
            pltpu.VMEM_SHARED((n_rows, width), _F32),
        ],
    )
    def sc_scatter(feat_hbm, idx_hbm, out_hbm, idx_refs, feat_v, zbuf, acc):
        c = lax.axis_index("c")
        s = lax.axis_index("s")
        wid = c * NS + s
        base = wid * chunk

        def zbody(i, carry):
            for j in range(width // 16):
                zbuf[i, pl.ds(j * 16, 16)] = jnp.zeros((16,), _F32)
            return carry

        lax.fori_loop(0, rows_per_sub, zbody, 0)
        pltpu.sync_copy(zbuf, acc.at[pl.ds(s * rows_per_sub, rows_per_sub)])
        for j in range(n_j):
            pltpu.sync_copy(idx_hbm.at[pl.ds(base + j * 128, 128)], idx_refs[j])
        pltpu.sync_copy(feat_hbm.at[pl.ds(base, chunk)], feat_v)
        plsc.subcore_barrier()
        for j in range(n_j):
            pltpu.sync_copy(feat_v.at[pl.ds(j * 128, 128)],
                            acc.at[idx_refs[j]], add=True)
        plsc.subcore_barrier()
        wb = c * (n_rows // NC) + s * wb_per_sub
        pltpu.sync_copy(acc.at[pl.ds(wb, wb_per_sub)],
                        out_hbm.at[pl.ds(wb, wb_per_sub)])

    return sc_scatter


def _res_kernel(c_tile_ref, cT_ref, attr_ref, gath_ref, Waa_ref, baa_ref,
                W1_ref, b1_ref, g_ref, be_ref, W2_ref, b2_ref, out_ref,
                *, TR, L, K, DPAD):
    c_tile = c_tile_ref[0]
    cT = cT_ref[0]
    attr = attr_ref[0]
    gath = gath_ref[0][:, :52]

    emb_aa = jax.lax.dot_general(attr, Waa_ref[...], (((1,), (0,)), ((), ())),
                                 precision=_DEF) + baa_ref[...]
    emb = jnp.concatenate([emb_aa, gath], axis=1)

    feats = _top4_gather(c_tile, cT, emb, L, K)
    feats.append(jnp.zeros((c_tile.shape[0], DPAD - 276), _F32))
    feat = jnp.concatenate(feats, axis=1)

    h = jax.lax.dot_general(feat, W1_ref[...], (((1,), (0,)), ((), ())),
                            precision=_DEF) + b1_ref[...]
    mu = jnp.mean(h, axis=1, keepdims=True)
    var = jnp.mean((h - mu) ** 2, axis=1, keepdims=True)
    hn = (h - mu) / jnp.sqrt(var + 1e-5) * g_ref[...] + be_ref[...]
    ge = 0.5 * hn * (1.0 + _erf(hn * 0.7071067811865476))
    logits = jax.lax.dot_general(ge, W2_ref[...], (((1,), (0,)), ((), ())),
                                 precision=_DEF) + b2_ref[...]
    out_ref[...] = logits[None]


def kernel(coord_aa, attr_aa, triplets_aa, indices_aa, coord_atom, attr_atom,
           triplets_atom, indices_atom, W_aa, b_aa, atom_table, W1, b1, gamma,
           beta, W2, b2):
    B, L, _ = coord_aa.shape
    La = coord_atom.shape[1]
    K = 4
    TA = 1024
    TR = 1024
    DPAD = 384
    WID = 128

    c_atom = coord_atom.astype(_F32)
    c_atomT = jnp.transpose(c_atom, (0, 2, 1))
    attr_f = attr_atom.astype(_F32)[:, None, :]
    idxres_f = indices_atom[..., 0].astype(_F32)[:, None, :]
    tblT = atom_table.T.astype(_F32)

    atom_feat, idx_g = pl.pallas_call(
        functools.partial(_atom_kernel, TA=TA, La=La, L=L, K=K),
        grid=(B, La // TA),
        in_specs=[
            pl.BlockSpec((1, TA, 3), lambda b, t: (b, t, 0)),
            pl.BlockSpec((1, 3, La), lambda b, t: (b, 0, 0)),
            pl.BlockSpec((1, 1, La), lambda b, t: (b, 0, 0)),
            pl.BlockSpec((1, 1, TA), lambda b, t: (b, 0, t)),
            pl.BlockSpec((12, 13), lambda b, t: (0, 0)),
        ],
        out_specs=[
            pl.BlockSpec((1, TA, WID), lambda b, t: (b, t, 0)),
            pl.BlockSpec((1, 1, TA), lambda b, t: (b, 0, t)),
        ],
        out_shape=[
            jax.ShapeDtypeStruct((B, La, WID), _F32),
            jax.ShapeDtypeStruct((B, 1, La), jnp.int32),
        ],
    )(c_atom, c_atomT, attr_f, idxres_f, tblT)

    n_tot = B * La
    sc_scatter = _make_sc_scatter(B * L, n_tot, WID, n_tot // 32)
    gathered = sc_scatter(atom_feat.reshape(n_tot, WID),
                          idx_g.reshape(n_tot))
    gathered = gathered.reshape(B, L, WID)

    c_aa = coord_aa.astype(_F32)
    c_aaT = jnp.transpose(c_aa, (0, 2, 1))
    W1p = W1[:DPAD].astype(_F32)

    out3 = pl.pallas_call(
        functools.partial(_res_kernel, TR=TR, L=L, K=K, DPAD=DPAD),
        grid=(B, L // TR),
        in_specs=[
            pl.BlockSpec((1, TR, 3), lambda b, t: (b, t, 0)),
            pl.BlockSpec((1, 3, L), lambda b, t: (b, 0, 0)),
            pl.BlockSpec((1, L, 20), lambda b, t: (b, 0, 0)),
            pl.BlockSpec((1, L, WID), lambda b, t: (b, 0, 0)),
            pl.BlockSpec((20, 16), lambda b, t: (0, 0)),
            pl.BlockSpec((1, 16), lambda b, t: (0, 0)),
            pl.BlockSpec((DPAD, 256), lambda b, t: (0, 0)),
            pl.BlockSpec((1, 256), lambda b, t: (0, 0)),
            pl.BlockSpec((1, 256), lambda b, t: (0, 0)),
            pl.BlockSpec((1, 256), lambda b, t: (0, 0)),
            pl.BlockSpec((256, 1), lambda b, t: (0, 0)),
            pl.BlockSpec((1, 1), lambda b, t: (0, 0)),
        ],
        out_specs=pl.BlockSpec((1, TR, 1), lambda b, t: (b, t, 0)),
        out_shape=jax.ShapeDtypeStruct((B, L, 1), _F32),
    )(c_aa, c_aaT, attr_aa.astype(_F32), gathered, W_aa.astype(_F32),
      b_aa.astype(_F32)[None], W1p, b1.astype(_F32)[None],
      gamma.astype(_F32)[None], beta.astype(_F32)[None], W2.astype(_F32),
      b2.astype(_F32)[None])

    return out3[..., 0]
